# baseline (speedup 1.0000x reference)
import dataclasses
import functools

import jax
import jax.numpy as jnp
from jax import lax
from jax.experimental import pallas as pl
from jax.experimental.pallas import tpu as pltpu
from jax.experimental.pallas import tpu_sc as plsc

NUM_CORES = 2
NUM_SUBCORES = 16
NW = NUM_CORES * NUM_SUBCORES
LANES = 16

BATCH = 16384
DIM = 64
B_PER_W = BATCH // NW
I_PER_W = 2 * B_PER_W


def _mf_kernel(ids_hbm, vals_hbm, table_hbm, out_hbm,
               idx_v, rows_v, vals_v, v01_v, out_v, part_v, sem):
    wid = lax.axis_index("s") * NUM_CORES + lax.axis_index("c")
    base = wid * B_PER_W

    pltpu.sync_copy(ids_hbm.at[pl.ds(2 * base, I_PER_W)], idx_v)
    pltpu.sync_copy(vals_hbm.at[pl.ds(2 * base, I_PER_W)], vals_v)

    gather = pltpu.async_copy(table_hbm.at[idx_v], rows_v, sem)

    lane_iota = lax.iota(jnp.int32, LANES)

    @pl.loop(0, B_PER_W, step=LANES)
    def _(g):
        even = 2 * (g + lane_iota)
        ve = plsc.load_gather(vals_v, [even])
        vo = plsc.load_gather(vals_v, [even + 1])
        v01_v[pl.ds(g, LANES)] = ve * vo

    gather.wait()

    @pl.loop(0, B_PER_W, step=LANES)
    def _(g):
        for r in range(LANES):
            b2 = 2 * (g + r)
            part = rows_v[b2, pl.ds(0, LANES)] * rows_v[b2 + 1, pl.ds(0, LANES)]
            for d in range(LANES, DIM, LANES):
                part += rows_v[b2, pl.ds(d, LANES)] * rows_v[b2 + 1, pl.ds(d, LANES)]
            part_v[r, pl.ds(0, LANES)] = part
        acc = plsc.load_gather(part_v, [lane_iota, jnp.full((LANES,), 0, jnp.int32)])
        for c in range(1, LANES):
            acc += plsc.load_gather(part_v, [lane_iota, jnp.full((LANES,), c, jnp.int32)])
        sl = pl.ds(g, LANES)
        out_v[sl] = acc * v01_v[sl]

    pltpu.sync_copy(out_v, out_hbm.at[pl.ds(base, B_PER_W)])


@jax.jit
def kernel(feature_ids, feature_vals, table):
    ids_flat = feature_ids.reshape(-1)
    vals_flat = feature_vals.reshape(-1)

    mesh = plsc.VectorSubcoreMesh(core_axis_name="c", subcore_axis_name="s")
    cp = pltpu.CompilerParams()
    for fld, val in (("needs_layout_passes", False),
                     ("use_tc_tiling_on_sc", False)):
        if fld in pltpu.CompilerParams.__dataclass_fields__:
            cp = dataclasses.replace(cp, **{fld: val})
    run = functools.partial(
        pl.kernel,
        mesh=mesh,
        compiler_params=cp,
        out_type=jax.ShapeDtypeStruct((BATCH,), jnp.float32),
        scratch_types=[
            pltpu.VMEM((I_PER_W,), jnp.int32),
            pltpu.VMEM((I_PER_W, DIM), jnp.float32),
            pltpu.VMEM((I_PER_W,), jnp.float32),
            pltpu.VMEM((B_PER_W,), jnp.float32),
            pltpu.VMEM((B_PER_W,), jnp.float32),
            pltpu.VMEM((LANES, LANES), jnp.float32),
            pltpu.SemaphoreType.DMA,
        ],
    )(_mf_kernel)
    return run(ids_flat, vals_flat, table)
